# baseline (speedup 1.0000x reference)
import jax
import jax.numpy as jnp
from jax import lax
from jax.experimental import pallas as pl
from jax.experimental.pallas import tpu as pltpu
from jax.experimental.pallas import tpu_sc as plsc

N = 10000
E = 320000
HID = 128
NC = 2
NS = 16
NPAD = 10240
NPS = NPAD // NS
C = 80
NCH = 250
EPT = NCH * C
EPAD = NS * EPT
L = 16
G = 10
NG = NCH // G


def _rsqrt_newton(d):
    i = plsc.bitcast(d, jnp.int32)
    i = jnp.int32(0x5F3759DF) - (i >> 1)
    y = plsc.bitcast(i, jnp.float32)
    for _ in range(4):
        y = y * (1.5 - 0.5 * d * y * y)
    return jnp.where(d > 0, y, 0.0)


def _sc_body(vsplit, e4, w4, out, degparts, dinvall, srcg, dstg, wg,
             nodebuf, redb, rowb, rowb2, dloc, sadjA, sadjB,
             normA, normB, accsh, sem, gsemA, gsemB, ssem):
    ci = lax.axis_index("c")
    si = lax.axis_index("s")

    def _zero_deg(k, _):
        nodebuf[pl.ds(k * L, L)] = jnp.zeros((L,), jnp.float32)
        return 0
    lax.fori_loop(0, NPAD // L, _zero_deg, 0)

    def _deg_group(g, _):
        d1 = pltpu.async_copy(e4.at[0, si, g], srcg, sem)
        d2 = pltpu.async_copy(w4.at[si, g], wg, sem)
        d1.wait(); d2.wait()

        def _deg_chunk(ch, _):
            for j in range(C // L):
                sidx = srcg[ch, pl.ds(j * L, L)]
                wval = wg[ch, pl.ds(j * L, L)]
                plsc.addupdate_scatter(nodebuf, [sidx], wval)
            return 0
        lax.fori_loop(0, G, _deg_chunk, 0)
        return 0
    lax.fori_loop(0, NG, _deg_group, 0)

    pltpu.sync_copy(nodebuf, degparts.at[ci, si])
    plsc.subcore_barrier()

    STR = 128
    for t in range(NPS // STR):
        pltpu.sync_copy(
            degparts.at[ci, :, pl.ds(si * NPS + t * STR, STR)], redb)

        def _dinv_vec(j, _):
            d = redb[0, pl.ds(j * L, L)]
            for p in range(1, NS):
                d = d + redb[p, pl.ds(j * L, L)]
            dloc[pl.ds(j * L, L)] = _rsqrt_newton(d)
            return 0
        lax.fori_loop(0, STR // L, _dinv_vec, 0)
        pltpu.sync_copy(dloc,
                        dinvall.at[ci, pl.ds(si * NPS + t * STR, STR)])
    plsc.subcore_barrier()
    pltpu.sync_copy(dinvall.at[ci], nodebuf)

    def _zero_row(r, _):
        for j in range(HID // L):
            rowb[r, pl.ds(j * L, L)] = jnp.zeros((L,), jnp.float32)
            rowb2[r, pl.ds(j * L, L)] = jnp.zeros((L,), jnp.float32)
        return 0
    lax.fori_loop(0, C, _zero_row, 0)
    ZR = 80
    zdescs = []
    for t in range(NPS // ZR):
        zb = rowb if t % 2 == 0 else rowb2
        zdescs.append(pltpu.async_copy(
            zb.at[pl.ds(0, ZR), :],
            accsh.at[pl.ds(si * NPS + t * ZR, ZR), :], sem))
    for d in zdescs:
        d.wait()
    plsc.subcore_barrier()

    coff = ci * N
    bufs = (rowb, rowb2)
    sadjs = (sadjA, sadjB)
    norms = (normA, normB)
    gsems = (gsemA, gsemB)

    def _prep(ch, sadj, normb):
        for j in range(C // L):
            sidx = srcg[ch, pl.ds(j * L, L)]
            didx = dstg[ch, pl.ds(j * L, L)]
            wval = wg[ch, pl.ds(j * L, L)]
            nsrc = plsc.load_gather(nodebuf, [sidx])
            ndst = plsc.load_gather(nodebuf, [didx])
            normb[pl.ds(j * L, L)] = -(nsrc * wval * ndst)
            sadj[pl.ds(j * L, L)] = sidx + coff

    def _gather(k):
        pltpu.async_copy(vsplit.at[sadjs[k]], bufs[k], gsems[k])

    def _drain_g(k):
        pltpu.make_async_copy(vsplit.at[pl.ds(0, C)], rowb, gsems[k]).wait()

    def _scale(k):
        rb = bufs[k]
        normb = norms[k]

        @plsc.parallel_loop(0, C, unroll=4)
        def _row(r):
            nb = plsc.load_gather(normb, [jnp.full((L,), r, jnp.int32)])
            for j in range(HID // L):
                rb[r, pl.ds(j * L, L)] = rb[r, pl.ds(j * L, L)] * nb

    def _scatter(k, ch):
        pltpu.async_copy(bufs[k], accsh.at[dstg.at[ch]], ssem, add=True)

    def _drain_s():
        pltpu.make_async_copy(vsplit.at[pl.ds(0, C)], rowb, ssem).wait()

    def _group(g, _):
        c1 = pltpu.async_copy(e4.at[0, si, g], srcg, sem)
        c2 = pltpu.async_copy(e4.at[1, si, g], dstg, sem)
        c3 = pltpu.async_copy(w4.at[si, g], wg, sem)
        c1.wait(); c2.wait(); c3.wait()

        _prep(0, sadjA, normA)
        _gather(0)
        _prep(1, sadjB, normB)
        _gather(1)

        def _pair(q, _):
            for k in range(2):
                c = 2 * q + k
                _drain_g(k)
                _scale(k)
                _scatter(k, c)

                @pl.when(q < G // 2 - 1)
                def _():
                    _prep(c + 2, sadjs[k], norms[k])
                _drain_s()
                @pl.when(q < G // 2 - 1)
                def _():
                    _gather(k)
            return 0
        lax.fori_loop(0, G // 2, _pair, 0)
        return 0
    lax.fori_loop(0, NG, _group, 0)

    plsc.subcore_barrier()
    pltpu.sync_copy(accsh.at[pl.ds(si * NPS, NPS), :],
                    out.at[ci, pl.ds(si * NPS, NPS), :])


def _sc_aggregate(x, h, edge_index, w):
    vsplit = jnp.concatenate([x, h], axis=0)
    pad = EPAD - E
    pidx = jnp.arange(pad, dtype=jnp.int32) % N
    ep = jnp.concatenate([edge_index, jnp.stack([pidx, pidx])], axis=1)
    wp = jnp.concatenate([w, jnp.zeros((pad,), jnp.float32)])
    e4 = ep.reshape(2, NS, NG, G, C)
    w4 = wp.reshape(NS, NG, G, C)
    mesh = plsc.VectorSubcoreMesh(core_axis_name="c", subcore_axis_name="s",
                                  num_cores=NC, num_subcores=NS)
    f = pl.kernel(
        _sc_body,
        out_type=[
            jax.ShapeDtypeStruct((NC, NPAD, HID), jnp.float32),
            jax.ShapeDtypeStruct((NC, NS, NPAD), jnp.float32),
            jax.ShapeDtypeStruct((NC, NPAD), jnp.float32),
        ],
        mesh=mesh,
        scratch_types=[
            pltpu.VMEM((G, C), jnp.int32),
            pltpu.VMEM((G, C), jnp.int32),
            pltpu.VMEM((G, C), jnp.float32),
            pltpu.VMEM((NPAD,), jnp.float32),
            pltpu.VMEM((NS, 128), jnp.float32),
            pltpu.VMEM((C, HID), jnp.float32),
            pltpu.VMEM((C, HID), jnp.float32),
            pltpu.VMEM((128,), jnp.float32),
            pltpu.VMEM((C,), jnp.int32),
            pltpu.VMEM((C,), jnp.int32),
            pltpu.VMEM((C,), jnp.float32),
            pltpu.VMEM((C,), jnp.float32),
            pltpu.VMEM_SHARED((NPAD, HID), jnp.float32),
            pltpu.SemaphoreType.DMA,
            pltpu.SemaphoreType.DMA,
            pltpu.SemaphoreType.DMA,
            pltpu.SemaphoreType.DMA,
        ],
        compiler_params=pltpu.CompilerParams(needs_layout_passes=False),
    )
    S, _, _ = f(vsplit, e4, w4)
    return S


R = 2000


def _tc_body(x_ref, h_ref, s_ref, w_ref, b_ref, wp_ref, c_ref, hn_ref, cn_ref):
    W = w_ref[...]
    g = (jnp.dot(x_ref[...], W[0:HID], preferred_element_type=jnp.float32)
         + jnp.dot(h_ref[...], W[HID:2 * HID], preferred_element_type=jnp.float32)
         + jnp.dot(s_ref[0], W[2 * HID:3 * HID], preferred_element_type=jnp.float32)
         + jnp.dot(s_ref[1], W[3 * HID:4 * HID], preferred_element_type=jnp.float32)
         + b_ref[...])
    cc = c_ref[...]
    gi = jax.nn.sigmoid(g[:, 0:HID] + wp_ref[0:1, :] * cc)
    gf = jax.nn.sigmoid(g[:, HID:2 * HID] + wp_ref[1:2, :] * cc)
    gt = jnp.tanh(g[:, 2 * HID:3 * HID])
    cn = gf * cc + gi * gt
    go = jax.nn.sigmoid(g[:, 3 * HID:4 * HID] + wp_ref[2:3, :] * cn)
    hn_ref[...] = go * jnp.tanh(cn)
    cn_ref[...] = cn


def _tc_gates(x, h, S, Wbig, bias, w_peep, c):
    hn, cn = pl.pallas_call(
        _tc_body,
        grid=(N // R,),
        in_specs=[
            pl.BlockSpec((R, HID), lambda i: (i, 0)),
            pl.BlockSpec((R, HID), lambda i: (i, 0)),
            pl.BlockSpec((2, R, HID), lambda i: (0, i, 0)),
            pl.BlockSpec((4 * HID, 4 * HID), lambda i: (0, 0)),
            pl.BlockSpec((1, 4 * HID), lambda i: (0, 0)),
            pl.BlockSpec((3, HID), lambda i: (0, 0)),
            pl.BlockSpec((R, HID), lambda i: (i, 0)),
        ],
        out_specs=[
            pl.BlockSpec((R, HID), lambda i: (i, 0)),
            pl.BlockSpec((R, HID), lambda i: (i, 0)),
        ],
        out_shape=[
            jax.ShapeDtypeStruct((N, HID), jnp.float32),
            jax.ShapeDtypeStruct((N, HID), jnp.float32),
        ],
    )(x, h, S, Wbig, bias, w_peep, c)
    return hn, cn


def kernel(x, edge_index, edge_weight, h, c, Wx, bx, Wh, bh, w_peep, b_gate):
    S = _sc_aggregate(x, h, edge_index, edge_weight)

    Wbig = jnp.concatenate([Wx[:, 0], Wh[:, 0], Wx[:, 1], Wh[:, 1]],
                           axis=1)
    Wbig = jnp.transpose(Wbig, (1, 0, 2)).reshape(4 * HID, 4 * HID)
    bias = (bx + bh + b_gate).reshape(1, 4 * HID)

    hn, cn = _tc_gates(x, h, S, Wbig, bias, w_peep, c)
    return (hn, hn, cn)


# speedup vs baseline: 1.1013x; 1.1013x over previous
import jax
import jax.numpy as jnp
from jax import lax
from jax.experimental import pallas as pl
from jax.experimental.pallas import tpu as pltpu
from jax.experimental.pallas import tpu_sc as plsc

N = 10000
E = 320000
HID = 128
NC = 2
NS = 16
NPAD = 10240
NPS = NPAD // NS
C = 80
NCH = 250
EPT = NCH * C
EPAD = NS * EPT
L = 16
G = 25
NG = NCH // G


def _rsqrt_newton(d):
    i = plsc.bitcast(d, jnp.int32)
    i = jnp.int32(0x5F3759DF) - (i >> 1)
    y = plsc.bitcast(i, jnp.float32)
    for _ in range(4):
        y = y * (1.5 - 0.5 * d * y * y)
    return jnp.where(d > 0, y, 0.0)


def _sc_body(vsplit, e4, w4, out, degparts, dinvall, srcg, dstg, wg,
             nodebuf, redb, rowb, rowb2, dloc, sadjA, sadjB,
             normA, normB, accsh, sem, gsemA, gsemB, ssem):
    ci = lax.axis_index("c")
    si = lax.axis_index("s")

    def _zero_deg(k, _):
        nodebuf[pl.ds(k * L, L)] = jnp.zeros((L,), jnp.float32)
        return 0
    lax.fori_loop(0, NPAD // L, _zero_deg, 0)

    def _deg_group(g, _):
        d1 = pltpu.async_copy(e4.at[0, si, g], srcg, sem)
        d2 = pltpu.async_copy(w4.at[si, g], wg, sem)
        d1.wait(); d2.wait()

        def _deg_chunk(ch, _):
            for j in range(C // L):
                sidx = srcg[ch, pl.ds(j * L, L)]
                wval = wg[ch, pl.ds(j * L, L)]
                plsc.addupdate_scatter(nodebuf, [sidx], wval)
            return 0
        lax.fori_loop(0, G, _deg_chunk, 0)
        return 0
    lax.fori_loop(0, NG, _deg_group, 0)

    pltpu.sync_copy(nodebuf, degparts.at[ci, si])
    plsc.subcore_barrier()

    STR = 128
    for t in range(NPS // STR):
        pltpu.sync_copy(
            degparts.at[ci, :, pl.ds(si * NPS + t * STR, STR)], redb)

        def _dinv_vec(j, _):
            d = redb[0, pl.ds(j * L, L)]
            for p in range(1, NS):
                d = d + redb[p, pl.ds(j * L, L)]
            dloc[pl.ds(j * L, L)] = _rsqrt_newton(d)
            return 0
        lax.fori_loop(0, STR // L, _dinv_vec, 0)
        pltpu.sync_copy(dloc,
                        dinvall.at[ci, pl.ds(si * NPS + t * STR, STR)])
    plsc.subcore_barrier()
    pltpu.sync_copy(dinvall.at[ci], nodebuf)

    def _zero_row(r, _):
        for j in range(HID // L):
            rowb[r, pl.ds(j * L, L)] = jnp.zeros((L,), jnp.float32)
            rowb2[r, pl.ds(j * L, L)] = jnp.zeros((L,), jnp.float32)
        return 0
    lax.fori_loop(0, C, _zero_row, 0)
    ZR = 80
    zdescs = []
    for t in range(NPS // ZR):
        zb = rowb if t % 2 == 0 else rowb2
        zdescs.append(pltpu.async_copy(
            zb.at[pl.ds(0, ZR), :],
            accsh.at[pl.ds(si * NPS + t * ZR, ZR), :], sem))
    for d in zdescs:
        d.wait()
    plsc.subcore_barrier()

    coff = ci * N
    bufs = (rowb, rowb2)
    sadjs = (sadjA, sadjB)
    norms = (normA, normB)
    gsems = (gsemA, gsemB)

    def _prep(ch, sadj, normb):
        for j in range(C // L):
            sidx = srcg[ch, pl.ds(j * L, L)]
            didx = dstg[ch, pl.ds(j * L, L)]
            wval = wg[ch, pl.ds(j * L, L)]
            nsrc = plsc.load_gather(nodebuf, [sidx])
            ndst = plsc.load_gather(nodebuf, [didx])
            normb[pl.ds(j * L, L)] = -(nsrc * wval * ndst)
            sadj[pl.ds(j * L, L)] = sidx + coff

    def _gather(k):
        pltpu.async_copy(vsplit.at[sadjs[k]], bufs[k], gsems[k])

    def _drain_g(k):
        pltpu.make_async_copy(vsplit.at[pl.ds(0, C)], rowb, gsems[k]).wait()

    def _scale(k):
        rb = bufs[k]
        normb = norms[k]

        @plsc.parallel_loop(0, C, unroll=4)
        def _row(r):
            nb = plsc.load_gather(normb, [jnp.full((L,), r, jnp.int32)])
            for j in range(HID // L):
                rb[r, pl.ds(j * L, L)] = rb[r, pl.ds(j * L, L)] * nb

    def _scatter(k, ch):
        pltpu.async_copy(bufs[k], accsh.at[dstg.at[ch]], ssem, add=True)

    def _drain_s():
        pltpu.make_async_copy(vsplit.at[pl.ds(0, C)], rowb, ssem).wait()

    def _group(g, _):
        c1 = pltpu.async_copy(e4.at[0, si, g], srcg, sem)
        c2 = pltpu.async_copy(e4.at[1, si, g], dstg, sem)
        c3 = pltpu.async_copy(w4.at[si, g], wg, sem)
        c1.wait(); c2.wait(); c3.wait()

        _prep(0, sadjA, normA)
        _gather(0)
        _prep(1, sadjB, normB)
        _gather(1)

        def _pair(p, _):
            c0 = 2 * p
            _drain_g(0)
            _scale(0)
            _scatter(0, c0)
            _prep(c0 + 2, sadjA, normA)
            _drain_s()
            _gather(0)

            _drain_g(1)
            _scale(1)
            _scatter(1, c0 + 1)

            @pl.when(p < (G - 3) // 2)
            def _():
                _prep(c0 + 3, sadjB, normB)
                _drain_s()
                _gather(1)
            return 0
        lax.fori_loop(0, (G - 1) // 2, _pair, 0)

        _drain_g(0)
        _scale(0)
        _scatter(0, G - 1)
        _drain_s()
        _drain_s()
        return 0
    lax.fori_loop(0, NG, _group, 0)

    plsc.subcore_barrier()
    pltpu.sync_copy(accsh.at[pl.ds(si * NPS, NPS), :],
                    out.at[ci, pl.ds(si * NPS, NPS), :])


def _sc_aggregate(x, h, edge_index, w):
    vsplit = jnp.concatenate([x, h], axis=0)
    pad = EPAD - E
    pidx = jnp.arange(pad, dtype=jnp.int32) % N
    ep = jnp.concatenate([edge_index, jnp.stack([pidx, pidx])], axis=1)
    wp = jnp.concatenate([w, jnp.zeros((pad,), jnp.float32)])
    e4 = ep.reshape(2, NS, NG, G, C)
    w4 = wp.reshape(NS, NG, G, C)
    mesh = plsc.VectorSubcoreMesh(core_axis_name="c", subcore_axis_name="s",
                                  num_cores=NC, num_subcores=NS)
    f = pl.kernel(
        _sc_body,
        out_type=[
            jax.ShapeDtypeStruct((NC, NPAD, HID), jnp.float32),
            jax.ShapeDtypeStruct((NC, NS, NPAD), jnp.float32),
            jax.ShapeDtypeStruct((NC, NPAD), jnp.float32),
        ],
        mesh=mesh,
        scratch_types=[
            pltpu.VMEM((G, C), jnp.int32),
            pltpu.VMEM((G, C), jnp.int32),
            pltpu.VMEM((G, C), jnp.float32),
            pltpu.VMEM((NPAD,), jnp.float32),
            pltpu.VMEM((NS, 128), jnp.float32),
            pltpu.VMEM((C, HID), jnp.float32),
            pltpu.VMEM((C, HID), jnp.float32),
            pltpu.VMEM((128,), jnp.float32),
            pltpu.VMEM((C,), jnp.int32),
            pltpu.VMEM((C,), jnp.int32),
            pltpu.VMEM((C,), jnp.float32),
            pltpu.VMEM((C,), jnp.float32),
            pltpu.VMEM_SHARED((NPAD, HID), jnp.float32),
            pltpu.SemaphoreType.DMA,
            pltpu.SemaphoreType.DMA,
            pltpu.SemaphoreType.DMA,
            pltpu.SemaphoreType.DMA,
        ],
        compiler_params=pltpu.CompilerParams(needs_layout_passes=False),
    )
    S, _, _ = f(vsplit, e4, w4)
    return S


R = 2000


def _tc_body(x_ref, h_ref, s_ref, w_ref, b_ref, wp_ref, c_ref, hn_ref, cn_ref):
    W = w_ref[...]
    g = (jnp.dot(x_ref[...], W[0:HID], preferred_element_type=jnp.float32)
         + jnp.dot(h_ref[...], W[HID:2 * HID], preferred_element_type=jnp.float32)
         + jnp.dot(s_ref[0], W[2 * HID:3 * HID], preferred_element_type=jnp.float32)
         + jnp.dot(s_ref[1], W[3 * HID:4 * HID], preferred_element_type=jnp.float32)
         + b_ref[...])
    cc = c_ref[...]
    gi = jax.nn.sigmoid(g[:, 0:HID] + wp_ref[0:1, :] * cc)
    gf = jax.nn.sigmoid(g[:, HID:2 * HID] + wp_ref[1:2, :] * cc)
    gt = jnp.tanh(g[:, 2 * HID:3 * HID])
    cn = gf * cc + gi * gt
    go = jax.nn.sigmoid(g[:, 3 * HID:4 * HID] + wp_ref[2:3, :] * cn)
    hn_ref[...] = go * jnp.tanh(cn)
    cn_ref[...] = cn


def _tc_gates(x, h, S, Wbig, bias, w_peep, c):
    hn, cn = pl.pallas_call(
        _tc_body,
        grid=(N // R,),
        in_specs=[
            pl.BlockSpec((R, HID), lambda i: (i, 0)),
            pl.BlockSpec((R, HID), lambda i: (i, 0)),
            pl.BlockSpec((2, R, HID), lambda i: (0, i, 0)),
            pl.BlockSpec((4 * HID, 4 * HID), lambda i: (0, 0)),
            pl.BlockSpec((1, 4 * HID), lambda i: (0, 0)),
            pl.BlockSpec((3, HID), lambda i: (0, 0)),
            pl.BlockSpec((R, HID), lambda i: (i, 0)),
        ],
        out_specs=[
            pl.BlockSpec((R, HID), lambda i: (i, 0)),
            pl.BlockSpec((R, HID), lambda i: (i, 0)),
        ],
        out_shape=[
            jax.ShapeDtypeStruct((N, HID), jnp.float32),
            jax.ShapeDtypeStruct((N, HID), jnp.float32),
        ],
    )(x, h, S, Wbig, bias, w_peep, c)
    return hn, cn


def kernel(x, edge_index, edge_weight, h, c, Wx, bx, Wh, bh, w_peep, b_gate):
    S = _sc_aggregate(x, h, edge_index, edge_weight)

    Wbig = jnp.concatenate([Wx[:, 0], Wh[:, 0], Wx[:, 1], Wh[:, 1]],
                           axis=1)
    Wbig = jnp.transpose(Wbig, (1, 0, 2)).reshape(4 * HID, 4 * HID)
    bias = (bx + bh + b_gate).reshape(1, 4 * HID)

    hn, cn = _tc_gates(x, h, S, Wbig, bias, w_peep, c)
    return (hn, hn, cn)
